# baseline (speedup 1.0000x reference)
import functools

import jax
import jax.numpy as jnp
from jax.experimental import pallas as pl
from jax.experimental.pallas import tpu as pltpu


def _stage_body(mask_ref, rw_ref, x_ref, w_ref, b_ref, o_ref,
                *, n_exp, relu_expert, relu_final):
    e = pl.program_id(1)
    y = jax.lax.dot_general(
        x_ref[...], w_ref[0],
        (((1,), (1,)), ((), ())),
        preferred_element_type=jnp.float32,
    )
    y = y + b_ref[0]
    if relu_expert:
        y = jnp.maximum(y, 0.0)
    m = mask_ref[0]
    coeff = (m[0].astype(jnp.float32) * rw_ref[:, 0]
             + m[1].astype(jnp.float32) * rw_ref[:, 1])
    contrib = y * coeff[:, None]

    @pl.when(e == 0)
    def _():
        o_ref[...] = contrib

    @pl.when(e != 0)
    def _():
        o_ref[...] = o_ref[...] + contrib

    if relu_final:
        @pl.when(e == n_exp - 1)
        def _():
            o_ref[...] = jnp.maximum(o_ref[...], 0.0)


def _stage(x, mask, rw, W, b, *, relu_expert, relu_final, tm):
    x = x.astype(jnp.bfloat16)
    W = W.astype(jnp.bfloat16)
    T, K = x.shape
    E, N, K2 = W.shape
    assert K == K2 and T % tm == 0
    nt = T // tm
    body = functools.partial(_stage_body, n_exp=E,
                             relu_expert=relu_expert, relu_final=relu_final)
    return pl.pallas_call(
        body,
        grid=(nt, E),
        in_specs=[
            pl.BlockSpec((1, 2, tm), lambda i, e: (e, 0, i)),
            pl.BlockSpec((tm, 2), lambda i, e: (i, 0)),
            pl.BlockSpec((tm, K), lambda i, e: (i, 0)),
            pl.BlockSpec((1, N, K), lambda i, e: (e, 0, 0)),
            pl.BlockSpec((1, 1, N), lambda i, e: (e, 0, 0)),
        ],
        out_specs=pl.BlockSpec((tm, N), lambda i, e: (i, 0)),
        out_shape=jax.ShapeDtypeStruct((T, N), jnp.float32),
        compiler_params=pltpu.CompilerParams(
            dimension_semantics=("parallel", "arbitrary"),
        ),
    )(mask, rw, x, W, b.reshape(E, 1, N))


def kernel(x, expert_mask1, expert_mask2, expert_mask3,
           routing_weights1, routing_weights2, routing_weights3,
           W1, b1, W2, b2, W3, b3):
    bsz, seq_len, hidden = x.shape
    T = bsz * seq_len
    xf = x.reshape(T, hidden)
    cs1 = _stage(xf, expert_mask1, routing_weights1, W1, b1,
                 relu_expert=False, relu_final=False, tm=512)
    cs2 = _stage(cs1, expert_mask2, routing_weights2, W2, b2,
                 relu_expert=False, relu_final=False, tm=512)
    out = _stage(cs2, expert_mask3, routing_weights3, W3, b3,
                 relu_expert=True, relu_final=True, tm=512)
    return out.reshape(bsz, seq_len, -1)


# speedup vs baseline: 1.1488x; 1.1488x over previous
import functools

import jax
import jax.numpy as jnp
from jax.experimental import pallas as pl
from jax.experimental.pallas import tpu as pltpu


def _stage_body(mask_ref, rw_ref, x_ref, w_ref, b_ref, o_ref,
                *, n_exp, relu_expert, relu_final):
    e = pl.program_id(1)
    y = jax.lax.dot_general(
        x_ref[...], w_ref[0],
        (((1,), (1,)), ((), ())),
        preferred_element_type=jnp.float32,
    )
    y = y + b_ref[0]
    if relu_expert:
        y = jnp.maximum(y, 0.0)
    m = mask_ref[0]
    coeff = (m[0].astype(jnp.float32) * rw_ref[:, 0]
             + m[1].astype(jnp.float32) * rw_ref[:, 1])
    contrib = y * coeff[:, None]

    @pl.when(e == 0)
    def _():
        o_ref[...] = contrib

    @pl.when(e != 0)
    def _():
        o_ref[...] = o_ref[...] + contrib

    if relu_final:
        @pl.when(e == n_exp - 1)
        def _():
            o_ref[...] = jnp.maximum(o_ref[...], 0.0)


def _stage(x, mask, rw, W, b, *, relu_expert, relu_final, tm):
    T, K = x.shape
    E, N, K2 = W.shape
    assert K == K2 and T % tm == 0
    nt = T // tm
    body = functools.partial(_stage_body, n_exp=E,
                             relu_expert=relu_expert, relu_final=relu_final)
    return pl.pallas_call(
        body,
        grid=(nt, E),
        in_specs=[
            pl.BlockSpec((1, 2, tm), lambda i, e: (e, 0, i)),
            pl.BlockSpec((tm, 2), lambda i, e: (i, 0)),
            pl.BlockSpec((tm, K), lambda i, e: (i, 0)),
            pl.BlockSpec((1, N, K), lambda i, e: (e, 0, 0)),
            pl.BlockSpec((1, 1, N), lambda i, e: (e, 0, 0)),
        ],
        out_specs=pl.BlockSpec((tm, N), lambda i, e: (i, 0)),
        out_shape=jax.ShapeDtypeStruct((T, N), jnp.float32),
        compiler_params=pltpu.CompilerParams(
            dimension_semantics=("parallel", "arbitrary"),
        ),
    )(mask, rw, x, W, b.reshape(E, 1, N))


def kernel(x, expert_mask1, expert_mask2, expert_mask3,
           routing_weights1, routing_weights2, routing_weights3,
           W1, b1, W2, b2, W3, b3):
    bsz, seq_len, hidden = x.shape
    T = bsz * seq_len
    xf = x.reshape(T, hidden)
    cs1 = _stage(xf, expert_mask1, routing_weights1, W1, b1,
                 relu_expert=False, relu_final=False, tm=512)
    cs2 = _stage(cs1, expert_mask2, routing_weights2, W2, b2,
                 relu_expert=False, relu_final=False, tm=512)
    out = _stage(cs2, expert_mask3, routing_weights3, W3, b3,
                 relu_expert=True, relu_final=True, tm=512)
    return out.reshape(bsz, seq_len, -1)


# speedup vs baseline: 1.4512x; 1.2632x over previous
import functools

import jax
import jax.numpy as jnp
from jax.experimental import pallas as pl
from jax.experimental.pallas import tpu as pltpu


def _stage_body(mask_ref, rw_ref, x_ref, w_ref, b_ref, o_ref,
                *, n_exp, relu_expert, relu_final):
    e = pl.program_id(1)
    y = jax.lax.dot_general(
        x_ref[...], w_ref[0],
        (((1,), (1,)), ((), ())),
        preferred_element_type=jnp.float32,
    )
    y = y + b_ref[0]
    if relu_expert:
        y = jnp.maximum(y, 0.0)
    m = mask_ref[0]
    coeff = (m[0].astype(jnp.float32) * rw_ref[:, 0]
             + m[1].astype(jnp.float32) * rw_ref[:, 1])
    contrib = y * coeff[:, None]

    @pl.when(e == 0)
    def _():
        o_ref[...] = contrib

    @pl.when(e != 0)
    def _():
        o_ref[...] = o_ref[...] + contrib

    if relu_final:
        @pl.when(e == n_exp - 1)
        def _():
            o_ref[...] = jnp.maximum(o_ref[...], 0.0)


def _stage(x, mask, rw, W, b, *, relu_expert, relu_final, tm):
    T, K = x.shape
    E, N, K2 = W.shape
    assert K == K2 and T % tm == 0
    nt = T // tm
    body = functools.partial(_stage_body, n_exp=E,
                             relu_expert=relu_expert, relu_final=relu_final)
    return pl.pallas_call(
        body,
        grid=(nt, E),
        in_specs=[
            pl.BlockSpec((1, 2, tm), lambda i, e: (e, 0, i)),
            pl.BlockSpec((tm, 2), lambda i, e: (i, 0)),
            pl.BlockSpec((tm, K), lambda i, e: (i, 0)),
            pl.BlockSpec((1, N, K), lambda i, e: (e, 0, 0)),
            pl.BlockSpec((1, 1, N), lambda i, e: (e, 0, 0)),
        ],
        out_specs=pl.BlockSpec((tm, N), lambda i, e: (i, 0)),
        out_shape=jax.ShapeDtypeStruct((T, N), jnp.float32),
        compiler_params=pltpu.CompilerParams(
            dimension_semantics=("parallel", "arbitrary"),
        ),
    )(mask, rw, x, W, b.reshape(E, 1, N))


def kernel(x, expert_mask1, expert_mask2, expert_mask3,
           routing_weights1, routing_weights2, routing_weights3,
           W1, b1, W2, b2, W3, b3):
    bsz, seq_len, hidden = x.shape
    T = bsz * seq_len
    xf = x.reshape(T, hidden)
    cs1 = _stage(xf, expert_mask1, routing_weights1, W1, b1,
                 relu_expert=False, relu_final=False, tm=2048)
    cs2 = _stage(cs1, expert_mask2, routing_weights2, W2, b2,
                 relu_expert=False, relu_final=False, tm=2048)
    out = _stage(cs2, expert_mask3, routing_weights3, W3, b3,
                 relu_expert=True, relu_final=True, tm=2048)
    return out.reshape(bsz, seq_len, -1)
